# baseline (speedup 1.0000x reference)
import functools

import numpy as np
import jax
import jax.numpy as jnp
from jax import lax
from jax.experimental import pallas as pl
from jax.experimental.pallas import tpu as pltpu
from jax.experimental.pallas import tpu_sc as plsc

_C, _H, _W, _B, _DIM = 3, 64, 64, 32, 64
_SEQ = _C * _H * _W
_KC = 128
_NCH = _SEQ // _KC
_LANES = 128


def _dest_perm() -> np.ndarray:
    diagonals = [[] for _ in range(_H + _W - 1)]
    for i in range(_H):
        for j in range(_W):
            s = i + j
            if s % 2 == 0:
                diagonals[s].insert(0, (i, j))
            else:
                diagonals[s].append((i, j))
    triples = []
    for diag in diagonals:
        for ij in diag:
            for c in range(_C):
                triples.append((c,) + ij)
    a = np.array(triples, dtype=np.int64)
    return ((a[:, 0] * _H + a[:, 1]) * _W + a[:, 2]).astype(np.int32)


def _scatter_rows() -> np.ndarray:
    d_of_k = _dest_perm()
    rows = np.arange(_B, dtype=np.int32)[:, None] * _SEQ + d_of_k[None, :]
    return rows.reshape(_B, _NCH, _KC)


_SCAT = _scatter_rows()


@functools.partial(
    pl.kernel,
    out_type=jax.ShapeDtypeStruct((_B * _SEQ, _LANES), jnp.float32),
    mesh=plsc.VectorSubcoreMesh(core_axis_name="c", subcore_axis_name="s"),
    scratch_types=[
        pltpu.VMEM((_NCH, _KC), jnp.int32),
        pltpu.VMEM((2, _DIM, _KC), jnp.float32),
        pltpu.VMEM((2, _KC, _LANES), jnp.float32),
        pltpu.SemaphoreType.DMA((2,)),
        pltpu.SemaphoreType.DMA((2,)),
    ],
    compiler_params=pltpu.CompilerParams(use_tc_tiling_on_sc=True, needs_layout_passes=False),
)
def _zigzag_scatter(xt_hbm, sidx_hbm, out_hbm, sidx_v, inb_v, stg_v, rsem, ssem):
    b = lax.axis_index("s") * 2 + lax.axis_index("c")

    pltpu.sync_copy(sidx_hbm.at[b], sidx_v)

    def read(kc, p):
        pltpu.async_copy(
            xt_hbm.at[b, :, pl.ds(kc * _KC, _KC)], inb_v.at[p], rsem.at[p]
        )

    def wait_read(p):
        pltpu.make_async_copy(
            xt_hbm.at[0, :, pl.ds(0, _KC)], inb_v.at[p], rsem.at[p]
        ).wait()

    def scatter(kc, p):
        pltpu.async_copy(stg_v.at[p], out_hbm.at[sidx_v.at[kc]], ssem.at[p])

    def drain_scatter(p):
        pltpu.make_async_copy(
            stg_v.at[p], out_hbm.at[pl.ds(0, _KC)], ssem.at[p]
        ).wait()

    lane = lax.iota(jnp.int32, 16)
    vrow = [lane + v0 for v0 in range(0, _DIM, 16)]
    jrow = [lane + j0 for j0 in range(0, _KC, 16)]

    def transpose(p):
        inb = inb_v.at[p]
        stg = stg_v.at[p]

        @plsc.parallel_loop(0, _DIM, unroll=16)
        def _(i):
            spl = jnp.full((16,), 0, jnp.int32) + i
            for vi in range(_DIM // 16):
                vals = plsc.load_gather(inb, [vrow[vi], spl])
                stg[i, pl.ds(vi * 16, 16)] = vals
            for j16 in range(_KC // 32, _KC // 16):
                vals = inb[i, pl.ds(j16 * 16, 16)]
                plsc.store_scatter(stg, [jrow[j16], spl], vals)

    read(0, 0)
    read(1, 1)

    def body(g, carry):
        for p in range(2):
            kc = g * 2 + p
            wait_read(p)

            @pl.when(g > 0)
            def _():
                drain_scatter(p)

            transpose(p)
            scatter(kc, p)

            @pl.when(g < _NCH // 2 - 1)
            def _():
                read(kc + 2, p)
        return carry

    lax.fori_loop(0, _NCH // 2, body, 0)

    for p in range(2):
        drain_scatter(p)


def kernel(x):
    xt = jnp.transpose(x, (1, 2, 0))
    out = _zigzag_scatter(xt, jnp.asarray(_SCAT))
    return (
        out.reshape(_B, _SEQ, _LANES)[:, :, :_DIM]
        .reshape(_B, _C, _H, _W, _DIM)
    )


# speedup vs baseline: 1.0210x; 1.0210x over previous
import functools

import numpy as np
import jax
import jax.numpy as jnp
from jax import lax
from jax.experimental import pallas as pl
from jax.experimental.pallas import tpu as pltpu
from jax.experimental.pallas import tpu_sc as plsc

_C, _H, _W, _B, _DIM = 3, 64, 64, 32, 64
_SEQ = _C * _H * _W
_KC = 128
_NCH = _SEQ // _KC
_LANES = 128


def _dest_perm() -> np.ndarray:
    diagonals = [[] for _ in range(_H + _W - 1)]
    for i in range(_H):
        for j in range(_W):
            s = i + j
            if s % 2 == 0:
                diagonals[s].insert(0, (i, j))
            else:
                diagonals[s].append((i, j))
    triples = []
    for diag in diagonals:
        for ij in diag:
            for c in range(_C):
                triples.append((c,) + ij)
    a = np.array(triples, dtype=np.int64)
    return ((a[:, 0] * _H + a[:, 1]) * _W + a[:, 2]).astype(np.int32)


def _scatter_rows() -> np.ndarray:
    d_of_k = _dest_perm()
    rows = np.arange(_B, dtype=np.int32)[:, None] * _SEQ + d_of_k[None, :]
    return rows.reshape(_B, _NCH, _KC)


_SCAT = _scatter_rows()


@functools.partial(
    pl.kernel,
    out_type=jax.ShapeDtypeStruct((_B * _SEQ, _LANES), jnp.float32),
    mesh=plsc.VectorSubcoreMesh(core_axis_name="c", subcore_axis_name="s"),
    scratch_types=[
        pltpu.VMEM((_NCH, _KC), jnp.int32),
        pltpu.VMEM((4, _DIM, _KC), jnp.float32),
        pltpu.VMEM((2, _KC, _LANES), jnp.float32),
        pltpu.SemaphoreType.DMA((4,)),
        pltpu.SemaphoreType.DMA((2,)),
    ],
    compiler_params=pltpu.CompilerParams(use_tc_tiling_on_sc=True, needs_layout_passes=False),
)
def _zigzag_scatter(xt_hbm, sidx_hbm, out_hbm, sidx_v, inb_v, stg_v, rsem, ssem):
    b = lax.axis_index("s") * 2 + lax.axis_index("c")

    pltpu.sync_copy(sidx_hbm.at[b], sidx_v)

    def read(kc, p):
        pltpu.async_copy(
            xt_hbm.at[b, :, pl.ds(kc * _KC, _KC)], inb_v.at[p], rsem.at[p]
        )

    def wait_read(p):
        pltpu.make_async_copy(
            xt_hbm.at[0, :, pl.ds(0, _KC)], inb_v.at[p], rsem.at[p]
        ).wait()

    def scatter(kc, p):
        pltpu.async_copy(stg_v.at[p], out_hbm.at[sidx_v.at[kc]], ssem.at[p])

    def drain_scatter(p):
        pltpu.make_async_copy(
            stg_v.at[p], out_hbm.at[pl.ds(0, _KC)], ssem.at[p]
        ).wait()

    lane = lax.iota(jnp.int32, 16)
    vrow = [lane + v0 for v0 in range(0, _DIM, 16)]
    jrow = [lane + j0 for j0 in range(0, _KC, 16)]

    def transpose2(pi, ps):
        inb = inb_v.at[pi]
        stg = stg_v.at[ps]

        @plsc.parallel_loop(0, _DIM, unroll=8)
        def _(i):
            spl = jnp.full((16,), 0, jnp.int32) + i
            for vi in range(_DIM // 16):
                vals = plsc.load_gather(inb, [vrow[vi], spl])
                stg[i, pl.ds(vi * 16, 16)] = vals
            for j16 in range(_KC // 32, _KC // 16):
                vals = inb[i, pl.ds(j16 * 16, 16)]
                plsc.store_scatter(stg, [jrow[j16], spl], vals)

    read(0, 0)
    read(1, 1)

    def body(g, carry):
        for q in range(4):
            kc = g * 4 + q
            ps = q % 2
            wait_read(q)

            if q < 2:
                read(kc + 2, (q + 2) % 4)

                @pl.when(g > 0)
                def _():
                    drain_scatter(ps)
            else:
                @pl.when(g < _NCH // 4 - 1)
                def _():
                    read(kc + 2, (q + 2) % 4)

                drain_scatter(ps)

            transpose2(q, ps)
            scatter(kc, ps)
        return carry

    lax.fori_loop(0, _NCH // 4, body, 0)

    for p in range(2):
        drain_scatter(p)


def kernel(x):
    xt = jnp.transpose(x, (1, 2, 0))
    out = _zigzag_scatter(xt, jnp.asarray(_SCAT))
    return (
        out.reshape(_B, _SEQ, _LANES)[:, :, :_DIM]
        .reshape(_B, _C, _H, _W, _DIM)
    )


# speedup vs baseline: 1.1617x; 1.1378x over previous
import functools

import numpy as np
import jax
import jax.numpy as jnp
from jax import lax
from jax.experimental import pallas as pl
from jax.experimental.pallas import tpu as pltpu
from jax.experimental.pallas import tpu_sc as plsc

_C, _H, _W, _B, _DIM = 3, 64, 64, 32, 64
_SEQ = _C * _H * _W
_KC = 128
_NCH = _SEQ // _KC
_LANES = 128


def _dest_perm() -> np.ndarray:
    diagonals = [[] for _ in range(_H + _W - 1)]
    for i in range(_H):
        for j in range(_W):
            s = i + j
            if s % 2 == 0:
                diagonals[s].insert(0, (i, j))
            else:
                diagonals[s].append((i, j))
    triples = []
    for diag in diagonals:
        for ij in diag:
            for c in range(_C):
                triples.append((c,) + ij)
    a = np.array(triples, dtype=np.int64)
    return ((a[:, 0] * _H + a[:, 1]) * _W + a[:, 2]).astype(np.int32)


def _scatter_rows() -> np.ndarray:
    d_of_k = _dest_perm()
    rows = np.arange(_B, dtype=np.int32)[:, None] * _SEQ + d_of_k[None, :]
    return rows.reshape(_B, _NCH, _KC)


_SCAT = _scatter_rows()


@functools.partial(
    pl.kernel,
    out_type=jax.ShapeDtypeStruct((_B * _SEQ, _LANES), jnp.float32),
    mesh=plsc.VectorSubcoreMesh(core_axis_name="c", subcore_axis_name="s"),
    scratch_types=[
        pltpu.VMEM((_NCH, _KC), jnp.int32),
        pltpu.VMEM((4, _DIM, _KC), jnp.float32),
        pltpu.VMEM((2, _KC, _LANES), jnp.float32),
        pltpu.SemaphoreType.DMA((4,)),
        pltpu.SemaphoreType.DMA((2,)),
    ],
    compiler_params=pltpu.CompilerParams(use_tc_tiling_on_sc=True, needs_layout_passes=False),
)
def _zigzag_scatter(xt_hbm, sidx_hbm, out_hbm, sidx_v, inb_v, stg_v, rsem, ssem):
    b = lax.axis_index("s") * 2 + lax.axis_index("c")

    pltpu.sync_copy(sidx_hbm.at[b], sidx_v)

    def read(kc, p):
        pltpu.async_copy(
            xt_hbm.at[b, :, pl.ds(kc * _KC, _KC)], inb_v.at[p], rsem.at[p]
        )

    def wait_read(p):
        pltpu.make_async_copy(
            xt_hbm.at[0, :, pl.ds(0, _KC)], inb_v.at[p], rsem.at[p]
        ).wait()

    def scatter(kc, p):
        pltpu.async_copy(stg_v.at[p], out_hbm.at[sidx_v.at[kc]], ssem.at[p])

    def drain_scatter(p):
        pltpu.make_async_copy(
            stg_v.at[p], out_hbm.at[pl.ds(0, _KC)], ssem.at[p]
        ).wait()

    lane = lax.iota(jnp.int32, 16)
    vrow = [lane + v0 for v0 in range(0, _DIM, 16)]
    jrow = [lane + j0 for j0 in range(0, _KC, 16)]

    def transpose2(pi, ps):
        inb = inb_v.at[pi]
        stg = stg_v.at[ps]

        @plsc.parallel_loop(0, _DIM, unroll=4)
        def _(i):
            spl = jnp.full((16,), 0, jnp.int32) + i
            for vi in range(_DIM // 16):
                vals = plsc.load_gather(inb, [vrow[vi], spl])
                stg[i, pl.ds(vi * 16, 16)] = vals
            for j16 in range(_KC // 32, _KC // 16):
                vals = inb[i, pl.ds(j16 * 16, 16)]
                plsc.store_scatter(stg, [jrow[j16], spl], vals)

    read(0, 0)
    read(1, 1)

    def body(g, carry):
        for q in range(4):
            kc = g * 4 + q
            ps = q % 2
            wait_read(q)

            if q < 2:
                read(kc + 2, (q + 2) % 4)

                @pl.when(g > 0)
                def _():
                    drain_scatter(ps)
            else:
                @pl.when(g < _NCH // 4 - 1)
                def _():
                    read(kc + 2, (q + 2) % 4)

                drain_scatter(ps)

            transpose2(q, ps)
            scatter(kc, ps)
        return carry

    lax.fori_loop(0, _NCH // 4, body, 0)

    for p in range(2):
        drain_scatter(p)


def kernel(x):
    xt = jnp.transpose(x, (1, 2, 0))
    out = _zigzag_scatter(xt, jnp.asarray(_SCAT))
    return (
        out.reshape(_B, _SEQ, _LANES)[:, :, :_DIM]
        .reshape(_B, _C, _H, _W, _DIM)
    )


# speedup vs baseline: 1.1923x; 1.0264x over previous
import functools

import numpy as np
import jax
import jax.numpy as jnp
from jax import lax
from jax.experimental import pallas as pl
from jax.experimental.pallas import tpu as pltpu
from jax.experimental.pallas import tpu_sc as plsc

_C, _H, _W, _B, _DIM = 3, 64, 64, 32, 64
_SEQ = _C * _H * _W
_KC = 128
_NCH = _SEQ // _KC
_LANES = 128


def _dest_perm() -> np.ndarray:
    diagonals = [[] for _ in range(_H + _W - 1)]
    for i in range(_H):
        for j in range(_W):
            s = i + j
            if s % 2 == 0:
                diagonals[s].insert(0, (i, j))
            else:
                diagonals[s].append((i, j))
    triples = []
    for diag in diagonals:
        for ij in diag:
            for c in range(_C):
                triples.append((c,) + ij)
    a = np.array(triples, dtype=np.int64)
    return ((a[:, 0] * _H + a[:, 1]) * _W + a[:, 2]).astype(np.int32)


def _scatter_rows() -> np.ndarray:
    d_of_k = _dest_perm()
    rows = np.arange(_B, dtype=np.int32)[:, None] * _SEQ + d_of_k[None, :]
    return rows.reshape(_B, _NCH, _KC)


_SCAT = _scatter_rows()


@functools.partial(
    pl.kernel,
    out_type=jax.ShapeDtypeStruct((_B * _SEQ, _LANES), jnp.float32),
    mesh=plsc.VectorSubcoreMesh(core_axis_name="c", subcore_axis_name="s"),
    scratch_types=[
        pltpu.VMEM((_NCH, _KC), jnp.int32),
        pltpu.VMEM((4, _DIM, _KC), jnp.float32),
        pltpu.VMEM((2, _KC, _LANES), jnp.float32),
        pltpu.SemaphoreType.DMA((4,)),
        pltpu.SemaphoreType.DMA((2,)),
    ],
    compiler_params=pltpu.CompilerParams(use_tc_tiling_on_sc=True, needs_layout_passes=False),
)
def _zigzag_scatter(xt_hbm, sidx_hbm, out_hbm, sidx_v, inb_v, stg_v, rsem, ssem):
    b = lax.axis_index("s") * 2 + lax.axis_index("c")

    pltpu.sync_copy(sidx_hbm.at[b], sidx_v)

    def read(kc, p):
        pltpu.async_copy(
            xt_hbm.at[b, :, pl.ds(kc * _KC, _KC)], inb_v.at[p], rsem.at[p]
        )

    def wait_read(p):
        pltpu.make_async_copy(
            xt_hbm.at[0, :, pl.ds(0, _KC)], inb_v.at[p], rsem.at[p]
        ).wait()

    def scatter(kc, p):
        pltpu.async_copy(stg_v.at[p], out_hbm.at[sidx_v.at[kc]], ssem.at[p])

    def drain_scatter(p):
        pltpu.make_async_copy(
            stg_v.at[p], out_hbm.at[pl.ds(0, _KC)], ssem.at[p]
        ).wait()

    lane = lax.iota(jnp.int32, 16)
    vrow = [lane + v0 for v0 in range(0, _DIM, 16)]
    jrow = [lane + j0 for j0 in range(0, _KC, 16)]

    def transpose2(pi, ps):
        inb = inb_v.at[pi]
        stg = stg_v.at[ps]

        @plsc.parallel_loop(0, _DIM, unroll=2)
        def _(i):
            spl = jnp.full((16,), 0, jnp.int32) + i
            for vi in range(_DIM // 16):
                vals = plsc.load_gather(inb, [vrow[vi], spl])
                stg[i, pl.ds(vi * 16, 16)] = vals
            for j16 in range(_KC // 32, _KC // 16):
                vals = inb[i, pl.ds(j16 * 16, 16)]
                plsc.store_scatter(stg, [jrow[j16], spl], vals)

    read(0, 0)
    read(1, 1)

    def body(g, carry):
        for q in range(4):
            kc = g * 4 + q
            ps = q % 2
            wait_read(q)

            if q < 2:
                read(kc + 2, (q + 2) % 4)

                @pl.when(g > 0)
                def _():
                    drain_scatter(ps)
            else:
                @pl.when(g < _NCH // 4 - 1)
                def _():
                    read(kc + 2, (q + 2) % 4)

                drain_scatter(ps)

            transpose2(q, ps)
            scatter(kc, ps)
        return carry

    lax.fori_loop(0, _NCH // 4, body, 0)

    for p in range(2):
        drain_scatter(p)


def kernel(x):
    xt = jnp.transpose(x, (1, 2, 0))
    out = _zigzag_scatter(xt, jnp.asarray(_SCAT))
    return (
        out.reshape(_B, _SEQ, _LANES)[:, :, :_DIM]
        .reshape(_B, _C, _H, _W, _DIM)
    )


# speedup vs baseline: 1.2694x; 1.0646x over previous
import functools

import numpy as np
import jax
import jax.numpy as jnp
from jax import lax
from jax.experimental import pallas as pl
from jax.experimental.pallas import tpu as pltpu
from jax.experimental.pallas import tpu_sc as plsc

_C, _H, _W, _B, _DIM = 3, 64, 64, 32, 64
_SEQ = _C * _H * _W
_KC = 128
_NCH = _SEQ // _KC
_LANES = 128


def _dest_perm() -> np.ndarray:
    diagonals = [[] for _ in range(_H + _W - 1)]
    for i in range(_H):
        for j in range(_W):
            s = i + j
            if s % 2 == 0:
                diagonals[s].insert(0, (i, j))
            else:
                diagonals[s].append((i, j))
    triples = []
    for diag in diagonals:
        for ij in diag:
            for c in range(_C):
                triples.append((c,) + ij)
    a = np.array(triples, dtype=np.int64)
    return ((a[:, 0] * _H + a[:, 1]) * _W + a[:, 2]).astype(np.int32)


def _scatter_rows() -> np.ndarray:
    d_of_k = _dest_perm()
    rows = np.arange(_B, dtype=np.int32)[:, None] * _SEQ + d_of_k[None, :]
    return rows.reshape(_B, _NCH, _KC)


_SCAT = _scatter_rows()


@functools.partial(
    pl.kernel,
    out_type=jax.ShapeDtypeStruct((_B * _SEQ, _LANES), jnp.float32),
    mesh=plsc.VectorSubcoreMesh(core_axis_name="c", subcore_axis_name="s"),
    scratch_types=[
        pltpu.VMEM((_NCH, _KC), jnp.int32),
        pltpu.VMEM((4, _DIM, _KC), jnp.float32),
        pltpu.VMEM((2, _KC, _LANES), jnp.float32),
        pltpu.SemaphoreType.DMA((4,)),
        pltpu.SemaphoreType.DMA((2,)),
    ],
    compiler_params=pltpu.CompilerParams(use_tc_tiling_on_sc=True, needs_layout_passes=False),
)
def _zigzag_scatter(xt_hbm, sidx_hbm, out_hbm, sidx_v, inb_v, stg_v, rsem, ssem):
    b = lax.axis_index("s") * 2 + lax.axis_index("c")

    pltpu.sync_copy(sidx_hbm.at[b], sidx_v)

    def read(kc, p):
        pltpu.async_copy(
            xt_hbm.at[b, :, pl.ds(kc * _KC, _KC)], inb_v.at[p], rsem.at[p]
        )

    def wait_read(p):
        pltpu.make_async_copy(
            xt_hbm.at[0, :, pl.ds(0, _KC)], inb_v.at[p], rsem.at[p]
        ).wait()

    def scatter(kc, p):
        pltpu.async_copy(stg_v.at[p], out_hbm.at[sidx_v.at[kc]], ssem.at[p])

    def drain_scatter(p):
        pltpu.make_async_copy(
            stg_v.at[p], out_hbm.at[pl.ds(0, _KC)], ssem.at[p]
        ).wait()

    lane = lax.iota(jnp.int32, 16)
    vrow = [lane + v0 for v0 in range(0, _DIM, 16)]
    jrow = [lane + j0 for j0 in range(0, _KC, 16)]

    def transpose2(pi, ps):
        inb = inb_v.at[pi]
        stg = stg_v.at[ps]

        @plsc.parallel_loop(0, _DIM, unroll=1)
        def _(i):
            spl = jnp.full((16,), 0, jnp.int32) + i
            for vi in range(_DIM // 16):
                vals = plsc.load_gather(inb, [vrow[vi], spl])
                stg[i, pl.ds(vi * 16, 16)] = vals
            for j16 in range(_KC // 32, _KC // 16):
                vals = inb[i, pl.ds(j16 * 16, 16)]
                plsc.store_scatter(stg, [jrow[j16], spl], vals)

    read(0, 0)
    read(1, 1)

    def body(g, carry):
        for q in range(4):
            kc = g * 4 + q
            ps = q % 2
            wait_read(q)

            if q < 2:
                read(kc + 2, (q + 2) % 4)

                @pl.when(g > 0)
                def _():
                    drain_scatter(ps)
            else:
                @pl.when(g < _NCH // 4 - 1)
                def _():
                    read(kc + 2, (q + 2) % 4)

                drain_scatter(ps)

            transpose2(q, ps)
            scatter(kc, ps)
        return carry

    lax.fori_loop(0, _NCH // 4, body, 0)

    for p in range(2):
        drain_scatter(p)


def kernel(x):
    xt = jnp.transpose(x, (1, 2, 0))
    out = _zigzag_scatter(xt, jnp.asarray(_SCAT))
    return (
        out.reshape(_B, _SEQ, _LANES)[:, :, :_DIM]
        .reshape(_B, _C, _H, _W, _DIM)
    )
